# baseline (speedup 1.0000x reference)
import functools

import jax
import jax.numpy as jnp
from jax import lax
from jax.experimental import pallas as pl
from jax.experimental.pallas import tpu as pltpu
from jax.experimental.pallas import tpu_sc as plsc

_BATCH = 16384
_FEAT = 64
_NC = 2
_NS = 16
_LANES = 16
_NW = _NC * _NS
_BPW = _BATCH // _NW
_GCHUNK = 128
_NGC = _BPW // _GCHUNK
_NCH = _FEAT // _LANES


def _center_loss_partials(features, labels, centers):
    mesh = plsc.VectorSubcoreMesh(core_axis_name="c", subcore_axis_name="s")

    @functools.partial(
        pl.kernel,
        out_type=jax.ShapeDtypeStruct((_NW, _LANES), jnp.float32),
        mesh=mesh,
        compiler_params=pltpu.CompilerParams(use_tc_tiling_on_sc=False),
        scratch_types=[
            pltpu.VMEM((_BPW,), jnp.int32),
            pltpu.VMEM((_BPW, _FEAT), jnp.float32),
            pltpu.VMEM((_BPW, _FEAT), jnp.float32),
            pltpu.VMEM((_NCH, _LANES), jnp.float32),
            pltpu.SemaphoreType.DMA,
            pltpu.SemaphoreType.DMA,
        ],
    )
    def k(feat_hbm, lab_hbm, cent_hbm, out_hbm,
          idx_v, rows_v, feats_v, acc_v, gsem, fsem):
        wid = lax.axis_index("s") * _NC + lax.axis_index("c")
        base = wid * _BPW

        pltpu.sync_copy(lab_hbm.at[pl.ds(base, _BPW)], idx_v)
        fcopy = pltpu.async_copy(feat_hbm.at[pl.ds(base, _BPW)], feats_v, fsem)
        for g in range(_NGC):
            pltpu.async_copy(
                cent_hbm.at[idx_v.at[pl.ds(g * _GCHUNK, _GCHUNK)]],
                rows_v.at[pl.ds(g * _GCHUNK, _GCHUNK)],
                gsem,
            )
        for g in range(_NGC):
            pltpu.make_async_copy(
                cent_hbm.at[idx_v.at[pl.ds(g * _GCHUNK, _GCHUNK)]],
                rows_v.at[pl.ds(g * _GCHUNK, _GCHUNK)],
                gsem,
            ).wait()
        fcopy.wait()

        for c in range(_NCH):
            acc_v[c] = jnp.zeros((_LANES,), jnp.float32)

        @pl.loop(0, _BPW)
        def _(r):
            for c in range(_NCH):
                d = (feats_v[r, pl.ds(c * _LANES, _LANES)]
                     - rows_v[r, pl.ds(c * _LANES, _LANES)])
                acc_v[c] += d * d

        total = acc_v[0] + acc_v[1] + acc_v[2] + acc_v[3]
        acc_v[0] = total
        pltpu.sync_copy(acc_v.at[0], out_hbm.at[wid])

    return k(features, labels, centers)


def kernel(features, labels, centers):
    partials = _center_loss_partials(features, labels.astype(jnp.int32), centers)
    return jnp.sum(partials) * (1.0 / _BATCH)


# speedup vs baseline: 1.7139x; 1.7139x over previous
import functools

import jax
import jax.numpy as jnp
from jax import lax
from jax.experimental import pallas as pl
from jax.experimental.pallas import tpu as pltpu
from jax.experimental.pallas import tpu_sc as plsc

_BATCH = 16384
_FEAT = 64
_NC = 2
_NS = 16
_LANES = 16
_NW = _NC * _NS
_BPW = _BATCH // _NW
_K = 16
_G = 4
_STEP = _G * _K
_NCH = _FEAT // _LANES


def _center_loss_partials(features, labels, centers):
    mesh = plsc.VectorSubcoreMesh(core_axis_name="c", subcore_axis_name="s")

    @functools.partial(
        pl.kernel,
        out_type=jax.ShapeDtypeStruct((_NW, _LANES), jnp.float32),
        mesh=mesh,
        scratch_types=[
            pltpu.VMEM((_BPW,), jnp.int32),
            pltpu.VMEM((_G * _K, _FEAT), jnp.float32),
            pltpu.VMEM((_BPW, _FEAT), jnp.float32),
            pltpu.VMEM((_NCH, _LANES), jnp.float32),
            pltpu.SemaphoreType.DMA,
        ] + [pltpu.SemaphoreType.DMA for _ in range(_G)],
    )
    def k(feat_hbm, lab_hbm, cent_hbm, out_hbm,
          idx_v, ring_v, feats_v, acc_v, fsem, *gsems):
        wid = lax.axis_index("s") * _NC + lax.axis_index("c")
        base = wid * _BPW

        pltpu.sync_copy(lab_hbm.at[pl.ds(base, _BPW)], idx_v)
        fcopy = pltpu.async_copy(feat_hbm.at[pl.ds(base, _BPW)], feats_v, fsem)

        def issue_group(slot, r0):
            rowvec = idx_v[pl.ds(r0, _K)]
            for j in range(_K):
                pltpu.async_copy(
                    cent_hbm.at[pl.ds(rowvec[j], 1)],
                    ring_v.at[pl.ds(slot * _K + j, 1)],
                    gsems[slot],
                )

        def drain_group(slot):
            for j in range(_K):
                pltpu.make_async_copy(
                    cent_hbm.at[pl.ds(0, 1)],
                    ring_v.at[pl.ds(slot * _K + j, 1)],
                    gsems[slot],
                ).wait()

        def compute_group(slot, r0):
            acc = [acc_v[c] for c in range(_NCH)]
            for j in range(_K):
                for c in range(_NCH):
                    d = (feats_v[r0 + j, pl.ds(c * _LANES, _LANES)]
                         - ring_v[slot * _K + j, pl.ds(c * _LANES, _LANES)])
                    acc[c] = acc[c] + d * d
            for c in range(_NCH):
                acc_v[c] = acc[c]

        for c in range(_NCH):
            acc_v[c] = jnp.zeros((_LANES,), jnp.float32)
        for slot in range(_G):
            issue_group(slot, slot * _K)
        fcopy.wait()

        @pl.loop(0, _BPW - _STEP, step=_STEP)
        def _(r0):
            for slot in range(_G):
                drain_group(slot)
                compute_group(slot, r0 + slot * _K)
                issue_group(slot, r0 + _STEP + slot * _K)

        for slot in range(_G):
            drain_group(slot)
            compute_group(slot, (_BPW - _STEP) + slot * _K)

        total = acc_v[0] + acc_v[1] + acc_v[2] + acc_v[3]
        acc_v[0] = total
        pltpu.sync_copy(acc_v.at[0], out_hbm.at[wid])

    return k(features, labels, centers)


def kernel(features, labels, centers):
    partials = _center_loss_partials(features, labels.astype(jnp.int32), centers)
    return jnp.sum(partials) * (1.0 / _BATCH)


# speedup vs baseline: 2.6087x; 1.5221x over previous
import dataclasses
import functools

import jax
import jax.numpy as jnp
from jax import lax
from jax.experimental import pallas as pl
from jax.experimental.pallas import tpu as pltpu
from jax.experimental.pallas import tpu_sc as plsc

_BATCH = 16384
_FEAT = 64
_NC = 2
_NS = 16
_LANES = 16
_NW = _NC * _NS
_BPW = _BATCH // _NW
_S = 4
_GRP = 16
_NGRP = _BPW // _GRP
_NACC = 4
_QCH = _FEAT // _LANES


def _center_loss_partials(features_t, labels, centers_t):
    mesh = plsc.VectorSubcoreMesh(core_axis_name="c", subcore_axis_name="s")

    @functools.partial(
        pl.kernel,
        out_type=jax.ShapeDtypeStruct((_NW, _LANES), jnp.float32),
        mesh=mesh,
        compiler_params=dataclasses.replace(
            pltpu.CompilerParams(), needs_layout_passes=False),
        scratch_types=[
            pltpu.VMEM((_BPW,), jnp.int32),
            pltpu.VMEM((_S, _FEAT, 128), jnp.float32),
            pltpu.VMEM((_FEAT, _BPW), jnp.float32),
            pltpu.VMEM((_FEAT * _GRP,), jnp.float32),
            pltpu.VMEM((_NACC, _LANES), jnp.float32),
            pltpu.SemaphoreType.DMA,
        ] + [pltpu.SemaphoreType.DMA for _ in range(_S)],
    )
    def k(feat_hbm, lab_hbm, cent_hbm, out_hbm,
          idx_v, ring_v, feats_v, stage_v, acc_v, fsem, *gsems):
        wid = lax.axis_index("s") * _NC + lax.axis_index("c")
        base = wid * _BPW

        pltpu.sync_copy(lab_hbm.at[pl.ds(base, _BPW)], idx_v)
        fcopy = pltpu.async_copy(feat_hbm.at[:, pl.ds(base, _BPW)], feats_v, fsem)

        iota = lax.iota(jnp.int32, _LANES)
        stage_base = [(iota + q * _LANES) * _GRP for q in range(_QCH)]

        def issue(slot, col):
            col_al = pl.multiple_of(jnp.bitwise_and(col, -128), 128)
            pltpu.async_copy(
                cent_hbm.at[:, pl.ds(col_al, 128)],
                ring_v.at[slot],
                gsems[slot],
            )

        def drain(slot):
            pltpu.make_async_copy(
                cent_hbm.at[:, pl.ds(0, 128)],
                ring_v.at[slot],
                gsems[slot],
            ).wait()

        def extract(slot, col, j):
            lane = jnp.broadcast_to(jnp.bitwise_and(col, 127), (_LANES,))
            jvec = jnp.broadcast_to(j, (_LANES,))
            for q in range(_QCH):
                v = plsc.load_gather(
                    ring_v.at[slot], [iota + q * _LANES, lane])
                plsc.store_scatter(stage_v, [stage_base[q] + jvec], v)

        def compute_group(r0):
            acc = [acc_v[a] for a in range(_NACC)]
            for d in range(_FEAT):
                dv = (feats_v[d, pl.ds(r0, _LANES)]
                      - stage_v[pl.ds(d * _GRP, _LANES)])
                a = d % _NACC
                acc[a] = acc[a] + dv * dv
            for a in range(_NACC):
                acc_v[a] = acc[a]

        for a in range(_NACC):
            acc_v[a] = jnp.zeros((_LANES,), jnp.float32)

        colvec0 = idx_v[pl.ds(0, _LANES)]
        for j in range(_S):
            issue(j, colvec0[j])
        fcopy.wait()

        @pl.loop(0, _NGRP - 1)
        def _(g):
            n0 = g * _GRP
            cur = idx_v[pl.ds(n0, _GRP)]
            nxt = idx_v[pl.ds(n0 + _GRP, _GRP)]
            for j in range(_GRP):
                slot = j % _S
                drain(slot)
                extract(slot, cur[j], j)
                issue(slot, cur[j + _S] if j < _GRP - _S else nxt[j - (_GRP - _S)])
            compute_group(n0)

        last0 = (_NGRP - 1) * _GRP
        cur = idx_v[pl.ds(last0, _GRP)]
        for j in range(_GRP):
            slot = j % _S
            drain(slot)
            extract(slot, cur[j], j)
            if j < _GRP - _S:
                issue(slot, cur[j + _S])
        compute_group(last0)

        total = acc_v[0] + acc_v[1] + acc_v[2] + acc_v[3]
        acc_v[0] = total
        pltpu.sync_copy(acc_v.at[0], out_hbm.at[wid])

    return k(features_t, labels, centers_t)


def kernel(features, labels, centers):
    partials = _center_loss_partials(
        features.T, labels.astype(jnp.int32), centers.T)
    return jnp.sum(partials) * (1.0 / _BATCH)


# speedup vs baseline: 2.7191x; 1.0423x over previous
import dataclasses
import functools

import jax
import jax.numpy as jnp
from jax import lax
from jax.experimental import pallas as pl
from jax.experimental.pallas import tpu as pltpu
from jax.experimental.pallas import tpu_sc as plsc

_BATCH = 16384
_FEAT = 64
_NC = 2
_NS = 16
_LANES = 16
_NW = _NC * _NS
_BPW = _BATCH // _NW
_S = 4
_GRP = 16
_NGRP = _BPW // _GRP
_NACC = 4
_QCH = _FEAT // _LANES


def _center_loss_partials(features_t, labels, centers_t):
    mesh = plsc.VectorSubcoreMesh(core_axis_name="c", subcore_axis_name="s")

    @functools.partial(
        pl.kernel,
        out_type=jax.ShapeDtypeStruct((_NW, _LANES), jnp.float32),
        mesh=mesh,
        compiler_params=dataclasses.replace(
            pltpu.CompilerParams(), needs_layout_passes=False),
        scratch_types=[
            pltpu.VMEM((_BPW,), jnp.int32),
            pltpu.VMEM((_S, _FEAT, 128), jnp.float32),
            pltpu.VMEM((_FEAT, _BPW), jnp.float32),
            pltpu.VMEM((_FEAT * _GRP,), jnp.float32),
            pltpu.VMEM((_NACC, _LANES), jnp.float32),
            pltpu.SemaphoreType.DMA,
        ] + [pltpu.SemaphoreType.DMA for _ in range(2 * _S)],
    )
    def k(feat_hbm, lab_hbm, cent_hbm, out_hbm,
          idx_v, ring_v, feats_v, stage_v, acc_v, fsem, *gsems):
        wid = lax.axis_index("s") * _NC + lax.axis_index("c")
        base = wid * _BPW

        pltpu.sync_copy(lab_hbm.at[pl.ds(base, _BPW)], idx_v)
        fcopy = pltpu.async_copy(feat_hbm.at[:, pl.ds(base, _BPW)], feats_v, fsem)

        iota = lax.iota(jnp.int32, _LANES)
        stage_base = [(iota + q * _LANES) * _GRP for q in range(_QCH)]

        def issue(slot, col):
            col_al = pl.multiple_of(jnp.bitwise_and(col, -128), 128)
            for h in range(2):
                pltpu.async_copy(
                    cent_hbm.at[pl.ds(h * 32, 32), pl.ds(col_al, 128)],
                    ring_v.at[slot, pl.ds(h * 32, 32)],
                    gsems[2 * slot + h],
                )

        def drain(slot):
            for h in range(2):
                pltpu.make_async_copy(
                    cent_hbm.at[pl.ds(h * 32, 32), pl.ds(0, 128)],
                    ring_v.at[slot, pl.ds(h * 32, 32)],
                    gsems[2 * slot + h],
                ).wait()

        def extract(slot, col, j):
            lane = jnp.broadcast_to(jnp.bitwise_and(col, 127), (_LANES,))
            jvec = jnp.broadcast_to(j, (_LANES,))
            for q in range(_QCH):
                v = plsc.load_gather(
                    ring_v.at[slot], [iota + q * _LANES, lane])
                plsc.store_scatter(stage_v, [stage_base[q] + jvec], v)

        def compute_group(r0):
            acc = [acc_v[a] for a in range(_NACC)]
            for d in range(_FEAT):
                dv = (feats_v[d, pl.ds(r0, _LANES)]
                      - stage_v[pl.ds(d * _GRP, _LANES)])
                a = d % _NACC
                acc[a] = acc[a] + dv * dv
            for a in range(_NACC):
                acc_v[a] = acc[a]

        for a in range(_NACC):
            acc_v[a] = jnp.zeros((_LANES,), jnp.float32)

        colvec0 = idx_v[pl.ds(0, _LANES)]
        for j in range(_S):
            issue(j, colvec0[j])
        fcopy.wait()

        @pl.loop(0, _NGRP - 1)
        def _(g):
            n0 = g * _GRP
            cur = idx_v[pl.ds(n0, _GRP)]
            nxt = idx_v[pl.ds(n0 + _GRP, _GRP)]
            for j in range(_GRP):
                slot = j % _S
                drain(slot)
                extract(slot, cur[j], j)
                issue(slot, cur[j + _S] if j < _GRP - _S else nxt[j - (_GRP - _S)])
            compute_group(n0)

        last0 = (_NGRP - 1) * _GRP
        cur = idx_v[pl.ds(last0, _GRP)]
        for j in range(_GRP):
            slot = j % _S
            drain(slot)
            extract(slot, cur[j], j)
            if j < _GRP - _S:
                issue(slot, cur[j + _S])
        compute_group(last0)

        total = acc_v[0] + acc_v[1] + acc_v[2] + acc_v[3]
        acc_v[0] = total
        pltpu.sync_copy(acc_v.at[0], out_hbm.at[wid])

    return k(features_t, labels, centers_t)


def kernel(features, labels, centers):
    partials = _center_loss_partials(
        features.T, labels.astype(jnp.int32), centers.T)
    return jnp.sum(partials) * (1.0 / _BATCH)
